# Initial kernel scaffold; baseline (speedup 1.0000x reference)
import functools

import jax
import jax.numpy as jnp
from jax import lax
from jax.experimental import pallas as pl
from jax.experimental.pallas import tpu as pltpu
from jax.experimental.pallas import tpu_sc as plsc

BM = 256
LMAX = 512


def _deg_body(a_ref, deg_ref):
    deg_ref[...] = jnp.sum(a_ref[...], axis=1, keepdims=True)


def _xw_body(x_ref, w_ref, deg_ref, u_ref):
    dis = lax.rsqrt(deg_ref[...] + 1.0)
    u_ref[...] = dis * jnp.dot(x_ref[...], w_ref[...],
                               preferred_element_type=jnp.float32)


def _conv_body(a_ref, uf_ref, ub_ref, deg_ref, b_ref, h_ref):
    dis = lax.rsqrt(deg_ref[...] + 1.0)
    acc = jnp.dot(a_ref[...], uf_ref[...],
                  preferred_element_type=jnp.float32) + ub_ref[...]
    h_ref[...] = jnp.maximum(dis * acc + b_ref[...], 0.0)


def _lx_body(a_ref, hf_ref, hb_ref, deg_ref, v_ref, xn_ref):
    w = jnp.dot(a_ref[...], hf_ref[...], preferred_element_type=jnp.float32)
    hb = hb_ref[...]
    lxv = deg_ref[...] * hb - w
    v_ref[...] = jnp.sqrt(jnp.sum(lxv * lxv, axis=1, keepdims=True) + 1e-12)
    xn_ref[...] = hb * lax.rsqrt(jnp.sum(hb * hb, axis=1, keepdims=True)
                                 + 1e-12)


def _nmax_body(a_ref, vc_ref, vb_ref, lead_ref):
    nm = jnp.max(a_ref[...] * vc_ref[...], axis=1, keepdims=True)
    lead_ref[...] = (vb_ref[...] >= nm).astype(jnp.int32)


def _cos_body(xn_ref, xl_ref, lead_ref, pos_ref, cnt_ref, s_ref):
    z = lax.dot_general(xn_ref[...], xl_ref[...],
                        (((1,), (1,)), ((), ())),
                        preferred_element_type=jnp.float32)
    m, n = z.shape
    colf = lax.broadcasted_iota(jnp.float32, (m, n), 1)
    z = jnp.where(colf < cnt_ref[...], z, -1e9)
    zmax = jnp.max(z, axis=1, keepdims=True)
    lo = zmax - 1.0
    hi = zmax

    def bis(_, lh):
        lo, hi = lh
        mid = 0.5 * (lo + hi)
        fs = jnp.sum(jnp.maximum(z - mid, 0.0), axis=1, keepdims=True)
        ok = fs >= 1.0
        return jnp.where(ok, mid, lo), jnp.where(ok, hi, mid)

    lo, hi = lax.fori_loop(0, 26, bis, (lo, hi))
    tau = lo
    for _ in range(2):
        sup = z > tau
        kk = jnp.sum(sup.astype(jnp.float32), axis=1, keepdims=True)
        ss = jnp.sum(jnp.where(sup, z, 0.0), axis=1, keepdims=True)
        tau = (ss - 1.0) / jnp.maximum(kk, 1.0)
    s_mat = jnp.maximum(z - tau, 0.0)
    coli = lax.broadcasted_iota(jnp.int32, (m, n), 1)
    onehot = (coli == pos_ref[...]).astype(jnp.float32)
    s_ref[...] = jnp.where(lead_ref[...] > 0, onehot, s_mat)


def _pool_body(a_ref, sf_ref, sb_ref, h_ref, ap_ref, xp_ref):
    i = pl.program_id(0)
    b_blk = jnp.dot(a_ref[...], sf_ref[...],
                    preferred_element_type=jnp.float32)
    ap = lax.dot_general(sb_ref[...], b_blk, (((0,), (0,)), ((), ())),
                         preferred_element_type=jnp.float32)
    xp = lax.dot_general(sb_ref[...], h_ref[...], (((0,), (0,)), ((), ())),
                         preferred_element_type=jnp.float32)

    @pl.when(i == 0)
    def _():
        ap_ref[...] = ap
        xp_ref[...] = xp

    @pl.when(i > 0)
    def _():
        ap_ref[...] += ap
        xp_ref[...] += xp


def _final_body(ap_ref, xp_ref, cnt_ref, w3_ref, b3_ref, w4_ref, b4_ref,
                wr1_ref, br1_ref, wr2_ref, br2_ref, wo_ref, bo_ref, out_ref):
    ap = ap_ref[...]
    l = ap.shape[0]
    rid = lax.broadcasted_iota(jnp.int32, (l, l), 0)
    cid = lax.broadcasted_iota(jnp.int32, (l, l), 1)
    ah = ap + (rid == cid).astype(jnp.float32)
    deg = jnp.sum(ah, axis=1, keepdims=True)
    dis = lax.rsqrt(deg)

    def conv(xv, w, b):
        u = dis * jnp.dot(xv, w, preferred_element_type=jnp.float32)
        t = jnp.dot(ah, u, preferred_element_type=jnp.float32)
        return jnp.maximum(dis * t + b, 0.0)

    x3 = conv(xp_ref[...], w3_ref[...], b3_ref[...])
    x4 = conv(x3, w4_ref[...], b4_ref[...])
    ridf = lax.broadcasted_iota(jnp.float32, (l, 1), 0)
    msk = ridf < cnt_ref[...]
    pooled = jnp.sum(jnp.where(msk, x4, 0.0), axis=0, keepdims=True)
    pooled = pooled / (cnt_ref[...] + 1e-12)
    h = jnp.dot(pooled, wr1_ref[...],
                preferred_element_type=jnp.float32) + br1_ref[...]
    h = jnp.dot(h, wr2_ref[...],
                preferred_element_type=jnp.float32) + br2_ref[...]
    out_ref[...] = jnp.dot(h, wo_ref[...],
                           preferred_element_type=jnp.float32) + bo_ref[...]


def _blk(bm, bn, imap):
    return pl.BlockSpec((bm, bn), imap)


def _row(i):
    return (i, 0)


def _fix(i):
    return (0, 0)


def _run_tc(n, f):
    g = n // BM
    f32 = jnp.float32

    def deg(a):
        return pl.pallas_call(
            _deg_body, grid=(g,),
            in_specs=[_blk(BM, n, _row)],
            out_specs=_blk(BM, 1, _row),
            out_shape=jax.ShapeDtypeStruct((n, 1), f32),
        )(a)

    def xw(x, w, degc):
        return pl.pallas_call(
            _xw_body, grid=(g,),
            in_specs=[_blk(BM, f, _row), _blk(f, f, _fix), _blk(BM, 1, _row)],
            out_specs=_blk(BM, f, _row),
            out_shape=jax.ShapeDtypeStruct((n, f), f32),
        )(x, w, degc)

    def conv(a, u, degc, b):
        return pl.pallas_call(
            _conv_body, grid=(g,),
            in_specs=[_blk(BM, n, _row), _blk(n, f, _fix), _blk(BM, f, _row),
                      _blk(BM, 1, _row), _blk(1, f, _fix)],
            out_specs=_blk(BM, f, _row),
            out_shape=jax.ShapeDtypeStruct((n, f), f32),
        )(a, u, u, degc, b)

    def lx(a, h, degc):
        return pl.pallas_call(
            _lx_body, grid=(g,),
            in_specs=[_blk(BM, n, _row), _blk(n, f, _fix), _blk(BM, f, _row),
                      _blk(BM, 1, _row)],
            out_specs=[_blk(BM, 1, _row), _blk(BM, f, _row)],
            out_shape=[jax.ShapeDtypeStruct((n, 1), f32),
                       jax.ShapeDtypeStruct((n, f), f32)],
        )(a, h, h, degc)

    def nmax(a, vcol, vblk):
        return pl.pallas_call(
            _nmax_body, grid=(g,),
            in_specs=[_blk(BM, n, _row), _blk(1, n, _fix), _blk(BM, 1, _row)],
            out_specs=_blk(BM, 1, _row),
            out_shape=jax.ShapeDtypeStruct((n, 1), jnp.int32),
        )(a, vcol, vblk)

    def cos(xn, xl, leadc, posc, cntf):
        return pl.pallas_call(
            _cos_body, grid=(g,),
            in_specs=[_blk(BM, f, _row), _blk(LMAX, f, _fix),
                      _blk(BM, 1, _row), _blk(BM, 1, _row), _blk(1, 1, _fix)],
            out_specs=_blk(BM, LMAX, _row),
            out_shape=jax.ShapeDtypeStruct((n, LMAX), f32),
        )(xn, xl, leadc, posc, cntf)

    def pool(a, s, h):
        return pl.pallas_call(
            _pool_body, grid=(g,),
            in_specs=[_blk(BM, n, _row), _blk(n, LMAX, _fix),
                      _blk(BM, LMAX, _row), _blk(BM, f, _row)],
            out_specs=[_blk(LMAX, LMAX, _fix), _blk(LMAX, f, _fix)],
            out_shape=[jax.ShapeDtypeStruct((LMAX, LMAX), f32),
                       jax.ShapeDtypeStruct((LMAX, f), f32)],
        )(a, s, s, h)

    def final(ap, xp, cntf, w3, b3, w4, b4, wr1, br1, wr2, br2, wo, bo):
        return pl.pallas_call(
            _final_body,
            in_specs=[_blk(LMAX, LMAX, _fix), _blk(LMAX, f, _fix),
                      _blk(1, 1, _fix), _blk(f, f, _fix), _blk(1, f, _fix),
                      _blk(f, f, _fix), _blk(1, f, _fix), _blk(f, f, _fix),
                      _blk(1, f, _fix), _blk(f, f, _fix), _blk(1, f, _fix),
                      _blk(f, 1, _fix), _blk(1, 1, _fix)],
            out_specs=_blk(1, 1, _fix),
            out_shape=jax.ShapeDtypeStruct((1, 1), f32),
        )(ap, xp, cntf, w3, b3, w4, b4, wr1, br1, wr2, br2, wo, bo)

    return deg, xw, conv, lx, nmax, cos, pool, final


def _sc_compact(leader, n):
    mesh = plsc.VectorSubcoreMesh(core_axis_name="c", subcore_axis_name="s")

    @functools.partial(
        pl.kernel, mesh=mesh,
        out_type=[jax.ShapeDtypeStruct((LMAX,), jnp.int32),
                  jax.ShapeDtypeStruct((n,), jnp.int32),
                  jax.ShapeDtypeStruct((16,), jnp.int32)],
        scratch_types=[pltpu.VMEM((n,), jnp.int32),
                       pltpu.VMEM((n,), jnp.int32),
                       pltpu.VMEM((LMAX,), jnp.int32),
                       pltpu.VMEM((16,), jnp.int32)],
    )
    def k(lead_hbm, idx_hbm, pos_hbm, cnt_hbm, flags_v, pos_v, idx_v, cnt_v):
        wid = lax.axis_index("s") * 2 + lax.axis_index("c")

        @pl.when(wid == 0)
        def _():
            pltpu.sync_copy(lead_hbm, flags_v)

            def zstep(t, carry):
                idx_v[pl.ds(t * 16, 16)] = jnp.zeros((16,), jnp.int32)
                return carry

            lax.fori_loop(0, LMAX // 16, zstep, 0)

            def step(t, carry):
                fl = flags_v[pl.ds(t * 16, 16)]
                cs = plsc.cumsum(fl)
                pos = carry + cs - 1
                pos_v[pl.ds(t * 16, 16)] = pos
                posc = jnp.minimum(jnp.maximum(pos, 0), LMAX - 1)
                vals = lax.iota(jnp.int32, 16) + t * 16
                plsc.store_scatter(idx_v, [posc], vals, mask=fl > 0)
                return carry + jnp.sum(fl)

            total = lax.fori_loop(0, n // 16, step, 0)
            cnt_v[...] = jnp.broadcast_to(total, (16,))
            pltpu.sync_copy(idx_v, idx_hbm)
            pltpu.sync_copy(pos_v, pos_hbm)
            pltpu.sync_copy(cnt_v, cnt_hbm)

    return k(leader)


def _sc_gather(lead_idx, xn):
    f = xn.shape[1]
    bpw = LMAX // 32
    mesh = plsc.VectorSubcoreMesh(core_axis_name="c", subcore_axis_name="s")

    @functools.partial(
        pl.kernel, mesh=mesh,
        out_type=jax.ShapeDtypeStruct((LMAX, f), jnp.float32),
        scratch_types=[pltpu.VMEM((bpw,), jnp.int32),
                       pltpu.VMEM((bpw, f), jnp.float32),
                       pltpu.SemaphoreType.DMA],
    )
    def k(idx_hbm, xn_hbm, out_hbm, idx_v, rows_v, sem):
        wid = lax.axis_index("s") * 2 + lax.axis_index("c")
        base = wid * bpw
        pltpu.sync_copy(idx_hbm.at[pl.ds(base, bpw)], idx_v)
        pltpu.async_copy(xn_hbm.at[idx_v], rows_v, sem).wait()
        pltpu.sync_copy(rows_v, out_hbm.at[pl.ds(base, bpw)])

    return k(lead_idx, xn)


def kernel(x, a, i, W1, b1, W2, b2, W3, b3, W4, b4,
           Wr1, br1, Wr2, br2, Wo, bo):
    n, f = x.shape
    deg_f, xw_f, conv_f, lx_f, nmax_f, cos_f, pool_f, final_f = _run_tc(n, f)

    b1r = b1.reshape(1, f)
    b2r = b2.reshape(1, f)

    deg = deg_f(a)
    u1 = xw_f(x, W1, deg)
    h1 = conv_f(a, u1, deg, b1r)
    u2 = xw_f(h1, W2, deg)
    h2 = conv_f(a, u2, deg, b2r)

    v, xn = lx_f(a, h2, deg)
    leader = nmax_f(a, v.reshape(1, n), v)

    lead_idx, pos, cnt = _sc_compact(leader.reshape(n), n)
    xl = _sc_gather(lead_idx, xn)

    cntf = cnt[:1].astype(jnp.float32).reshape(1, 1)
    s_mat = cos_f(xn, xl, leader, pos.reshape(n, 1), cntf)
    ap, xp = pool_f(a, s_mat, h2)

    out = final_f(ap, xp, cntf, W3, b3.reshape(1, f), W4, b4.reshape(1, f),
                  Wr1, br1.reshape(1, f), Wr2, br2.reshape(1, f),
                  Wo, bo.reshape(1, 1))
    return out


# speedup vs baseline: 6.7831x; 6.7831x over previous
import functools

import jax
import jax.numpy as jnp
from jax import lax
from jax.experimental import pallas as pl
from jax.experimental.pallas import tpu as pltpu
from jax.experimental.pallas import tpu_sc as plsc

BM = 256
LMAX = 512


def _deg_body(a_ref, deg_ref):
    deg_ref[...] = jnp.sum(a_ref[...], axis=1, keepdims=True)


def _xw_body(x_ref, w_ref, u_ref):
    u_ref[...] = jnp.dot(x_ref[...], w_ref[...],
                         preferred_element_type=jnp.float32)


def _eye_blk(bm, n):
    rowg = pl.program_id(0) * bm + lax.broadcasted_iota(jnp.int32, (bm, n), 0)
    colg = lax.broadcasted_iota(jnp.int32, (bm, n), 1)
    return rowg == colg


def _conv_body(a_ref, zf_ref, degr_ref, degc_ref, b_ref, h_ref):
    a = a_ref[...]
    bm, n = a.shape
    eye = _eye_blk(bm, n).astype(jnp.float32)
    dis_r = 1.0 / jnp.sqrt(degr_ref[...] + 1.0)
    dis_c = 1.0 / jnp.sqrt(degc_ref[...] + 1.0)
    lap = ((a + eye) * dis_r) * dis_c
    acc = jnp.dot(lap, zf_ref[...], preferred_element_type=jnp.float32)
    h_ref[...] = jnp.maximum(acc + b_ref[...], 0.0)


def _lx_body(a_ref, hf_ref, hb_ref, degr_ref, v_ref, xn_ref):
    a = a_ref[...]
    bm, n = a.shape
    eyem = _eye_blk(bm, n)
    lapc = jnp.where(eyem, degr_ref[...], 0.0) - a
    w = jnp.dot(lapc, hf_ref[...], preferred_element_type=jnp.float32)
    hb = hb_ref[...]
    v_ref[...] = jnp.sqrt(jnp.sum(w * w, axis=1, keepdims=True) + 1e-12)
    xn_ref[...] = hb / jnp.sqrt(jnp.sum(hb * hb, axis=1, keepdims=True)
                                + 1e-12)


def _nmax_body(a_ref, vc_ref, vb_ref, lead_ref):
    nm = jnp.max(a_ref[...] * vc_ref[...], axis=1, keepdims=True)
    lead_ref[...] = (vb_ref[...] >= nm).astype(jnp.int32)


def _cos_body(xn_ref, xl_ref, lead_ref, pos_ref, cnt_ref, s_ref):
    z = lax.dot_general(xn_ref[...], xl_ref[...],
                        (((1,), (1,)), ((), ())),
                        preferred_element_type=jnp.float32)
    m, n = z.shape
    coli = lax.broadcasted_iota(jnp.int32, (m, n), 1)
    z = jnp.where(coli.astype(jnp.float32) < cnt_ref[...], z, -1e9)
    zmax = jnp.max(z, axis=1, keepdims=True)
    lo = zmax - 1.0
    hi = zmax

    def bis(_, lh):
        lo, hi = lh
        mid = 0.5 * (lo + hi)
        fs = jnp.sum(jnp.maximum(z - mid, 0.0), axis=1, keepdims=True)
        ok = fs >= 1.0
        return jnp.where(ok, mid, lo), jnp.where(ok, hi, mid)

    lo, hi = lax.fori_loop(0, 26, bis, (lo, hi))
    tau = lo
    for _ in range(2):
        sup = z > tau
        kk = jnp.sum(sup.astype(jnp.float32), axis=1, keepdims=True)
        ss = jnp.sum(jnp.where(sup, z, 0.0), axis=1, keepdims=True)
        tau = (ss - 1.0) / jnp.maximum(kk, 1.0)
    s_mat = jnp.maximum(z - tau, 0.0)
    onehot = (coli == pos_ref[...]).astype(jnp.float32)
    s_ref[...] = jnp.where(lead_ref[...] > 0, onehot, s_mat)


def _t_body(s_ref, a_ref, t_ref):
    t_ref[...] = lax.dot_general(s_ref[...], a_ref[...],
                                 (((0,), (0,)), ((), ())),
                                 preferred_element_type=jnp.float32)


def _ap_body(t_ref, s_ref, h_ref, ap_ref, xp_ref, degp_ref):
    ap = jnp.dot(t_ref[...], s_ref[...], preferred_element_type=jnp.float32)
    ap_ref[...] = ap
    xp_ref[...] = lax.dot_general(s_ref[...], h_ref[...],
                                  (((0,), (0,)), ((), ())),
                                  preferred_element_type=jnp.float32)
    degp_ref[...] = jnp.sum(ap, axis=1, keepdims=True) + 1.0


def _final_body(ap_ref, xp_ref, degp_ref, degpc_ref, cnt_ref,
                w3_ref, b3_ref, w4_ref, b4_ref,
                wr1_ref, br1_ref, wr2_ref, br2_ref, wo_ref, bo_ref, out_ref):
    ap = ap_ref[...]
    l = ap.shape[0]
    rid = lax.broadcasted_iota(jnp.int32, (l, l), 0)
    cid = lax.broadcasted_iota(jnp.int32, (l, l), 1)
    ah = ap + (rid == cid).astype(jnp.float32)
    lap = ((ah * (1.0 / jnp.sqrt(degp_ref[...])))
           * (1.0 / jnp.sqrt(degpc_ref[...])))

    def conv(xv, w, b):
        z = jnp.dot(xv, w, preferred_element_type=jnp.float32)
        t = jnp.dot(lap, z, preferred_element_type=jnp.float32)
        return jnp.maximum(t + b, 0.0)

    x3 = conv(xp_ref[...], w3_ref[...], b3_ref[...])
    x4 = conv(x3, w4_ref[...], b4_ref[...])
    ridf = lax.broadcasted_iota(jnp.int32, (l, 1), 0).astype(jnp.float32)
    msk = ridf < cnt_ref[...]
    pooled = jnp.sum(jnp.where(msk, x4, 0.0), axis=0, keepdims=True)
    pooled = pooled / (cnt_ref[...] + 1e-12)
    h = jnp.dot(pooled, wr1_ref[...],
                preferred_element_type=jnp.float32) + br1_ref[...]
    h = jnp.dot(h, wr2_ref[...],
                preferred_element_type=jnp.float32) + br2_ref[...]
    out_ref[...] = jnp.dot(h, wo_ref[...],
                           preferred_element_type=jnp.float32) + bo_ref[...]


def _blk(bm, bn, imap):
    return pl.BlockSpec((bm, bn), imap)


def _row(i):
    return (i, 0)


def _fix(*_):
    return (0, 0)


def _run_tc(n, f):
    g = n // BM
    f32 = jnp.float32

    def deg(a):
        return pl.pallas_call(
            _deg_body, grid=(g,),
            in_specs=[_blk(BM, n, _row)],
            out_specs=_blk(BM, 1, _row),
            out_shape=jax.ShapeDtypeStruct((n, 1), f32),
        )(a)

    def xw(x, w):
        return pl.pallas_call(
            _xw_body, grid=(g,),
            in_specs=[_blk(BM, f, _row), _blk(f, f, _fix)],
            out_specs=_blk(BM, f, _row),
            out_shape=jax.ShapeDtypeStruct((n, f), f32),
        )(x, w)

    def conv(a, z, degr, degc, b):
        return pl.pallas_call(
            _conv_body, grid=(g,),
            in_specs=[_blk(BM, n, _row), _blk(n, f, _fix), _blk(BM, 1, _row),
                      _blk(1, n, _fix), _blk(1, f, _fix)],
            out_specs=_blk(BM, f, _row),
            out_shape=jax.ShapeDtypeStruct((n, f), f32),
        )(a, z, degr, degc, b)

    def lx(a, h, degr):
        return pl.pallas_call(
            _lx_body, grid=(g,),
            in_specs=[_blk(BM, n, _row), _blk(n, f, _fix), _blk(BM, f, _row),
                      _blk(BM, 1, _row)],
            out_specs=[_blk(BM, 1, _row), _blk(BM, f, _row)],
            out_shape=[jax.ShapeDtypeStruct((n, 1), f32),
                       jax.ShapeDtypeStruct((n, f), f32)],
        )(a, h, h, degr)

    def nmax(a, vcol, vblk):
        return pl.pallas_call(
            _nmax_body, grid=(g,),
            in_specs=[_blk(BM, n, _row), _blk(1, n, _fix), _blk(BM, 1, _row)],
            out_specs=_blk(BM, 1, _row),
            out_shape=jax.ShapeDtypeStruct((n, 1), jnp.int32),
        )(a, vcol, vblk)

    def cos(xn, xl, leadc, posc, cntf):
        return pl.pallas_call(
            _cos_body, grid=(g,),
            in_specs=[_blk(BM, f, _row), _blk(LMAX, f, _fix),
                      _blk(BM, 1, _row), _blk(BM, 1, _row), _blk(1, 1, _fix)],
            out_specs=_blk(BM, LMAX, _row),
            out_shape=jax.ShapeDtypeStruct((n, LMAX), f32),
        )(xn, xl, leadc, posc, cntf)

    def pool(a, s, h):
        t = pl.pallas_call(
            _t_body, grid=(g,),
            in_specs=[_blk(n, LMAX, _fix),
                      pl.BlockSpec((n, BM), lambda i: (0, i))],
            out_specs=pl.BlockSpec((LMAX, BM), lambda i: (0, i)),
            out_shape=jax.ShapeDtypeStruct((LMAX, n), f32),
        )(s, a)
        return pl.pallas_call(
            _ap_body,
            in_specs=[_blk(LMAX, n, _fix), _blk(n, LMAX, _fix),
                      _blk(n, f, _fix)],
            out_specs=[_blk(LMAX, LMAX, _fix), _blk(LMAX, f, _fix),
                       _blk(LMAX, 1, _fix)],
            out_shape=[jax.ShapeDtypeStruct((LMAX, LMAX), f32),
                       jax.ShapeDtypeStruct((LMAX, f), f32),
                       jax.ShapeDtypeStruct((LMAX, 1), f32)],
        )(t, s, h)

    def final(ap, xp, degp, degpc, cntf,
              w3, b3, w4, b4, wr1, br1, wr2, br2, wo, bo):
        return pl.pallas_call(
            _final_body,
            in_specs=[_blk(LMAX, LMAX, _fix), _blk(LMAX, f, _fix),
                      _blk(LMAX, 1, _fix), _blk(1, LMAX, _fix),
                      _blk(1, 1, _fix), _blk(f, f, _fix), _blk(1, f, _fix),
                      _blk(f, f, _fix), _blk(1, f, _fix), _blk(f, f, _fix),
                      _blk(1, f, _fix), _blk(f, f, _fix), _blk(1, f, _fix),
                      _blk(f, 1, _fix), _blk(1, 1, _fix)],
            out_specs=_blk(1, 1, _fix),
            out_shape=jax.ShapeDtypeStruct((1, 1), f32),
        )(ap, xp, degp, degpc, cntf,
          w3, b3, w4, b4, wr1, br1, wr2, br2, wo, bo)

    return deg, xw, conv, lx, nmax, cos, pool, final


def _sc_compact(leader, n):
    mesh = plsc.VectorSubcoreMesh(core_axis_name="c", subcore_axis_name="s")

    @functools.partial(
        pl.kernel, mesh=mesh,
        out_type=[jax.ShapeDtypeStruct((LMAX,), jnp.int32),
                  jax.ShapeDtypeStruct((n,), jnp.int32),
                  jax.ShapeDtypeStruct((16,), jnp.int32)],
        scratch_types=[pltpu.VMEM((n,), jnp.int32),
                       pltpu.VMEM((n,), jnp.int32),
                       pltpu.VMEM((LMAX,), jnp.int32),
                       pltpu.VMEM((16,), jnp.int32)],
        compiler_params=pltpu.CompilerParams(needs_layout_passes=False),
    )
    def k(lead_hbm, idx_hbm, pos_hbm, cnt_hbm, flags_v, pos_v, idx_v, cnt_v):
        wid = lax.axis_index("s") * 2 + lax.axis_index("c")

        @pl.when(wid == 0)
        def _():
            pltpu.sync_copy(lead_hbm, flags_v)

            def zstep(t, carry):
                idx_v[pl.ds(t * 16, 16)] = jnp.zeros((16,), jnp.int32)
                return carry

            lax.fori_loop(0, LMAX // 16, zstep, 0)

            def step(t, carry):
                fl = flags_v[pl.ds(t * 16, 16)]
                cs = plsc.cumsum(fl)
                pos = carry + cs - 1
                pos_v[pl.ds(t * 16, 16)] = pos
                posc = jnp.minimum(jnp.maximum(pos, 0), LMAX - 1)
                vals = lax.iota(jnp.int32, 16) + t * 16
                plsc.store_scatter(idx_v, [posc], vals, mask=fl > 0)
                return carry + jnp.sum(fl)

            total = lax.fori_loop(0, n // 16, step, 0)
            cnt_v[...] = jnp.broadcast_to(total, (16,))
            pltpu.sync_copy(idx_v, idx_hbm)
            pltpu.sync_copy(pos_v, pos_hbm)
            pltpu.sync_copy(cnt_v, cnt_hbm)

    return k(leader)


def _sc_gather(lead_idx, xn):
    f = xn.shape[1]
    bpw = LMAX // 32
    mesh = plsc.VectorSubcoreMesh(core_axis_name="c", subcore_axis_name="s")

    @functools.partial(
        pl.kernel, mesh=mesh,
        out_type=jax.ShapeDtypeStruct((LMAX, f), jnp.float32),
        scratch_types=[pltpu.VMEM((bpw,), jnp.int32),
                       pltpu.VMEM((bpw, f), jnp.float32),
                       pltpu.SemaphoreType.DMA],
    )
    def k(idx_hbm, xn_hbm, out_hbm, idx_v, rows_v, sem):
        wid = lax.axis_index("s") * 2 + lax.axis_index("c")
        base = wid * bpw
        pltpu.sync_copy(idx_hbm.at[pl.ds(base, bpw)], idx_v)
        pltpu.async_copy(xn_hbm.at[idx_v], rows_v, sem).wait()
        pltpu.sync_copy(rows_v, out_hbm.at[pl.ds(base, bpw)])

    return k(lead_idx, xn)


def kernel(x, a, i, W1, b1, W2, b2, W3, b3, W4, b4,
           Wr1, br1, Wr2, br2, Wo, bo):
    n, f = x.shape
    deg_f, xw_f, conv_f, lx_f, nmax_f, cos_f, pool_f, final_f = _run_tc(n, f)

    b1r = b1.reshape(1, f)
    b2r = b2.reshape(1, f)

    deg = deg_f(a)
    degc = deg.reshape(1, n)
    z1 = xw_f(x, W1)
    h1 = conv_f(a, z1, deg, degc, b1r)
    z2 = xw_f(h1, W2)
    h2 = conv_f(a, z2, deg, degc, b2r)

    v, xn = lx_f(a, h2, deg)
    leader = nmax_f(a, v.reshape(1, n), v)

    lead_idx, pos, cnt = _sc_compact(leader.reshape(n), n)
    xl = _sc_gather(lead_idx, xn)

    cntf = cnt[:1].astype(jnp.float32).reshape(1, 1)
    s_mat = cos_f(xn, xl, leader, pos.reshape(n, 1), cntf)
    ap, xp, degp = pool_f(a, s_mat, h2)

    out = final_f(ap, xp, degp, degp.reshape(1, LMAX), cntf,
                  W3, b3.reshape(1, f), W4, b4.reshape(1, f),
                  Wr1, br1.reshape(1, f), Wr2, br2.reshape(1, f),
                  Wo, bo.reshape(1, 1))
    return out


# speedup vs baseline: 9.2314x; 1.3609x over previous
import functools

import jax
import jax.numpy as jnp
from jax import lax
from jax.experimental import pallas as pl
from jax.experimental.pallas import tpu as pltpu
from jax.experimental.pallas import tpu_sc as plsc

BM = 256
LMAX = 256


def _degz_body(a_ref, x_ref, w_ref, deg_ref, z_ref):
    deg_ref[...] = jnp.sum(a_ref[...], axis=1, keepdims=True)
    z_ref[...] = jnp.dot(x_ref[...], w_ref[...],
                         preferred_element_type=jnp.float32)


def _eye_blk(bm, n):
    rowg = pl.program_id(0) * bm + lax.broadcasted_iota(jnp.int32, (bm, n), 0)
    colg = lax.broadcasted_iota(jnp.int32, (bm, n), 1)
    return rowg == colg


def _lap_blk(a, degr, degc):
    bm, n = a.shape
    eye = _eye_blk(bm, n).astype(jnp.float32)
    dis_r = 1.0 / jnp.sqrt(degr + 1.0)
    dis_c = 1.0 / jnp.sqrt(degc + 1.0)
    return ((a + eye) * dis_r) * dis_c


def _conv1_body(a_ref, zf_ref, degr_ref, degc_ref, b_ref, w2_ref,
                h_ref, z2_ref):
    lap = _lap_blk(a_ref[...], degr_ref[...], degc_ref[...])
    acc = jnp.dot(lap, zf_ref[...], preferred_element_type=jnp.float32)
    h = jnp.maximum(acc + b_ref[...], 0.0)
    h_ref[...] = h
    z2_ref[...] = jnp.dot(h, w2_ref[...], preferred_element_type=jnp.float32)


def _conv2_body(a_ref, zf_ref, degr_ref, degc_ref, b_ref, h_ref, xn_ref):
    lap = _lap_blk(a_ref[...], degr_ref[...], degc_ref[...])
    acc = jnp.dot(lap, zf_ref[...], preferred_element_type=jnp.float32)
    h = jnp.maximum(acc + b_ref[...], 0.0)
    h_ref[...] = h
    xn_ref[...] = h / jnp.sqrt(jnp.sum(h * h, axis=1, keepdims=True) + 1e-12)


def _lx_body(a_ref, hf_ref, degr_ref, v_ref):
    a = a_ref[...]
    bm, n = a.shape
    eyem = _eye_blk(bm, n)
    lapc = jnp.where(eyem, degr_ref[...], 0.0) - a
    w = jnp.dot(lapc, hf_ref[...], preferred_element_type=jnp.float32)
    v_ref[...] = jnp.sqrt(jnp.sum(w * w, axis=1, keepdims=True) + 1e-12)


def _nmax_body(a_ref, vc_ref, vb_ref, lead_ref):
    nm = jnp.max(a_ref[...] * vc_ref[...], axis=1, keepdims=True)
    lead_ref[...] = (vb_ref[...] >= nm).astype(jnp.int32)


def _cos_body(xn_ref, xl_ref, lead_ref, pos_ref, cnt_ref, s_ref):
    z = lax.dot_general(xn_ref[...], xl_ref[...],
                        (((1,), (1,)), ((), ())),
                        preferred_element_type=jnp.float32)
    m, n = z.shape
    coli = lax.broadcasted_iota(jnp.int32, (m, n), 1)
    z = jnp.where(coli.astype(jnp.float32) < cnt_ref[...], z, -1e9)
    zmax = jnp.max(z, axis=1, keepdims=True)
    lo = zmax - 1.0
    hi = zmax

    def bis(_, lh):
        lo, hi = lh
        mid = 0.5 * (lo + hi)
        fs = jnp.sum(jnp.maximum(z - mid, 0.0), axis=1, keepdims=True)
        ok = fs >= 1.0
        return jnp.where(ok, mid, lo), jnp.where(ok, hi, mid)

    lo, hi = lax.fori_loop(0, 14, bis, (lo, hi))
    tau = lo
    for _ in range(3):
        sup = z > tau
        kk = jnp.sum(sup.astype(jnp.float32), axis=1, keepdims=True)
        ss = jnp.sum(jnp.where(sup, z, 0.0), axis=1, keepdims=True)
        tau = (ss - 1.0) / jnp.maximum(kk, 1.0)
    s_mat = jnp.maximum(z - tau, 0.0)
    onehot = (coli == pos_ref[...]).astype(jnp.float32)
    s_ref[...] = jnp.where(lead_ref[...] > 0, onehot, s_mat)


def _t_body(s_ref, a_ref, t_ref):
    t_ref[...] = lax.dot_general(s_ref[...], a_ref[...],
                                 (((0,), (0,)), ((), ())),
                                 preferred_element_type=jnp.float32)


def _poolfinal_body(t_ref, s_ref, h_ref, cnt_ref,
                    w3_ref, b3_ref, w4_ref, b4_ref,
                    wr1_ref, br1_ref, wr2_ref, br2_ref, wo_ref, bo_ref,
                    out_ref):
    ap = jnp.dot(t_ref[...], s_ref[...], preferred_element_type=jnp.float32)
    xp = lax.dot_general(s_ref[...], h_ref[...], (((0,), (0,)), ((), ())),
                         preferred_element_type=jnp.float32)
    l = ap.shape[0]
    rid = lax.broadcasted_iota(jnp.int32, (l, l), 0)
    cid = lax.broadcasted_iota(jnp.int32, (l, l), 1)
    ah = ap + (rid == cid).astype(jnp.float32)
    deg_r = jnp.sum(ah, axis=1, keepdims=True)
    deg_c = jnp.sum(ah, axis=0, keepdims=True)
    lap = ((ah * (1.0 / jnp.sqrt(deg_r))) * (1.0 / jnp.sqrt(deg_c)))

    def conv(xv, w, b):
        z = jnp.dot(xv, w, preferred_element_type=jnp.float32)
        t = jnp.dot(lap, z, preferred_element_type=jnp.float32)
        return jnp.maximum(t + b, 0.0)

    x3 = conv(xp, w3_ref[...], b3_ref[...])
    x4 = conv(x3, w4_ref[...], b4_ref[...])
    ridf = lax.broadcasted_iota(jnp.int32, (l, 1), 0).astype(jnp.float32)
    msk = ridf < cnt_ref[...]
    pooled = jnp.sum(jnp.where(msk, x4, 0.0), axis=0, keepdims=True)
    pooled = pooled / (cnt_ref[...] + 1e-12)
    h = jnp.dot(pooled, wr1_ref[...],
                preferred_element_type=jnp.float32) + br1_ref[...]
    h = jnp.dot(h, wr2_ref[...],
                preferred_element_type=jnp.float32) + br2_ref[...]
    out_ref[...] = jnp.dot(h, wo_ref[...],
                           preferred_element_type=jnp.float32) + bo_ref[...]


def _blk(bm, bn, imap):
    return pl.BlockSpec((bm, bn), imap)


def _row(i):
    return (i, 0)


def _fix(*_):
    return (0, 0)


def _run_tc(n, f):
    g = n // BM
    f32 = jnp.float32

    def degz(a, x, w1):
        return pl.pallas_call(
            _degz_body, grid=(g,),
            in_specs=[_blk(BM, n, _row), _blk(BM, f, _row), _blk(f, f, _fix)],
            out_specs=[_blk(BM, 1, _row), _blk(BM, f, _row)],
            out_shape=[jax.ShapeDtypeStruct((n, 1), f32),
                       jax.ShapeDtypeStruct((n, f), f32)],
        )(a, x, w1)

    def conv1(a, z, degr, degc, b, w2):
        return pl.pallas_call(
            _conv1_body, grid=(g,),
            in_specs=[_blk(BM, n, _row), _blk(n, f, _fix), _blk(BM, 1, _row),
                      _blk(1, n, _fix), _blk(1, f, _fix), _blk(f, f, _fix)],
            out_specs=[_blk(BM, f, _row), _blk(BM, f, _row)],
            out_shape=[jax.ShapeDtypeStruct((n, f), f32),
                       jax.ShapeDtypeStruct((n, f), f32)],
        )(a, z, degr, degc, b, w2)

    def conv2(a, z, degr, degc, b):
        return pl.pallas_call(
            _conv2_body, grid=(g,),
            in_specs=[_blk(BM, n, _row), _blk(n, f, _fix), _blk(BM, 1, _row),
                      _blk(1, n, _fix), _blk(1, f, _fix)],
            out_specs=[_blk(BM, f, _row), _blk(BM, f, _row)],
            out_shape=[jax.ShapeDtypeStruct((n, f), f32),
                       jax.ShapeDtypeStruct((n, f), f32)],
        )(a, z, degr, degc, b)

    def lx(a, h, degr):
        return pl.pallas_call(
            _lx_body, grid=(g,),
            in_specs=[_blk(BM, n, _row), _blk(n, f, _fix), _blk(BM, 1, _row)],
            out_specs=_blk(BM, 1, _row),
            out_shape=jax.ShapeDtypeStruct((n, 1), f32),
        )(a, h, degr)

    def nmax(a, vcol, vblk):
        return pl.pallas_call(
            _nmax_body, grid=(g,),
            in_specs=[_blk(BM, n, _row), _blk(1, n, _fix), _blk(BM, 1, _row)],
            out_specs=_blk(BM, 1, _row),
            out_shape=jax.ShapeDtypeStruct((n, 1), jnp.int32),
        )(a, vcol, vblk)

    def cos(xn, xl, leadc, posc, cntf):
        return pl.pallas_call(
            _cos_body, grid=(g,),
            in_specs=[_blk(BM, f, _row), _blk(LMAX, f, _fix),
                      _blk(BM, 1, _row), _blk(BM, 1, _row), _blk(1, 1, _fix)],
            out_specs=_blk(BM, LMAX, _row),
            out_shape=jax.ShapeDtypeStruct((n, LMAX), f32),
        )(xn, xl, leadc, posc, cntf)

    def tpass(a, s):
        return pl.pallas_call(
            _t_body, grid=(g,),
            in_specs=[_blk(n, LMAX, _fix),
                      pl.BlockSpec((n, BM), lambda i: (0, i))],
            out_specs=pl.BlockSpec((LMAX, BM), lambda i: (0, i)),
            out_shape=jax.ShapeDtypeStruct((LMAX, n), f32),
        )(s, a)

    def poolfinal(t, s, h, cntf, w3, b3, w4, b4, wr1, br1, wr2, br2, wo, bo):
        return pl.pallas_call(
            _poolfinal_body,
            in_specs=[_blk(LMAX, n, _fix), _blk(n, LMAX, _fix),
                      _blk(n, f, _fix), _blk(1, 1, _fix),
                      _blk(f, f, _fix), _blk(1, f, _fix),
                      _blk(f, f, _fix), _blk(1, f, _fix), _blk(f, f, _fix),
                      _blk(1, f, _fix), _blk(f, f, _fix), _blk(1, f, _fix),
                      _blk(f, 1, _fix), _blk(1, 1, _fix)],
            out_specs=_blk(1, 1, _fix),
            out_shape=jax.ShapeDtypeStruct((1, 1), f32),
        )(t, s, h, cntf, w3, b3, w4, b4, wr1, br1, wr2, br2, wo, bo)

    return degz, conv1, conv2, lx, nmax, cos, tpass, poolfinal


def _sc_compact(leader, n):
    mesh = plsc.VectorSubcoreMesh(core_axis_name="c", subcore_axis_name="s")

    @functools.partial(
        pl.kernel, mesh=mesh,
        out_type=[jax.ShapeDtypeStruct((LMAX,), jnp.int32),
                  jax.ShapeDtypeStruct((n,), jnp.int32),
                  jax.ShapeDtypeStruct((16,), jnp.int32)],
        scratch_types=[pltpu.VMEM((n,), jnp.int32),
                       pltpu.VMEM((n,), jnp.int32),
                       pltpu.VMEM((LMAX,), jnp.int32),
                       pltpu.VMEM((16,), jnp.int32)],
        compiler_params=pltpu.CompilerParams(needs_layout_passes=False),
    )
    def k(lead_hbm, idx_hbm, pos_hbm, cnt_hbm, flags_v, pos_v, idx_v, cnt_v):
        wid = lax.axis_index("s") * 2 + lax.axis_index("c")

        @pl.when(wid == 0)
        def _():
            pltpu.sync_copy(lead_hbm, flags_v)

            def zstep(t, carry):
                idx_v[pl.ds(t * 16, 16)] = jnp.zeros((16,), jnp.int32)
                return carry

            lax.fori_loop(0, LMAX // 16, zstep, 0)

            def step(t, carry):
                fl = flags_v[pl.ds(t * 16, 16)]
                cs = plsc.cumsum(fl)
                pos = carry + cs - 1
                pos_v[pl.ds(t * 16, 16)] = pos
                posc = jnp.minimum(jnp.maximum(pos, 0), LMAX - 1)
                vals = lax.iota(jnp.int32, 16) + t * 16
                plsc.store_scatter(idx_v, [posc], vals, mask=fl > 0)
                return carry + jnp.sum(fl)

            total = lax.fori_loop(0, n // 16, step, 0)
            cnt_v[...] = jnp.broadcast_to(total, (16,))
            pltpu.sync_copy(idx_v, idx_hbm)
            pltpu.sync_copy(pos_v, pos_hbm)
            pltpu.sync_copy(cnt_v, cnt_hbm)

    return k(leader)


def _sc_gather(lead_idx, xn):
    f = xn.shape[1]
    bpw = LMAX // 32
    mesh = plsc.VectorSubcoreMesh(core_axis_name="c", subcore_axis_name="s")

    @functools.partial(
        pl.kernel, mesh=mesh,
        out_type=jax.ShapeDtypeStruct((LMAX, f), jnp.float32),
        scratch_types=[pltpu.VMEM((bpw,), jnp.int32),
                       pltpu.VMEM((bpw, f), jnp.float32),
                       pltpu.SemaphoreType.DMA],
    )
    def k(idx_hbm, xn_hbm, out_hbm, idx_v, rows_v, sem):
        wid = lax.axis_index("s") * 2 + lax.axis_index("c")
        base = wid * bpw
        pltpu.sync_copy(idx_hbm.at[pl.ds(base, bpw)], idx_v)
        pltpu.async_copy(xn_hbm.at[idx_v], rows_v, sem).wait()
        pltpu.sync_copy(rows_v, out_hbm.at[pl.ds(base, bpw)])

    return k(lead_idx, xn)


def kernel(x, a, i, W1, b1, W2, b2, W3, b3, W4, b4,
           Wr1, br1, Wr2, br2, Wo, bo):
    n, f = x.shape
    degz_f, conv1_f, conv2_f, lx_f, nmax_f, cos_f, tpass_f, poolfinal_f = \
        _run_tc(n, f)

    deg, z1 = degz_f(a, x, W1)
    degc = deg.reshape(1, n)
    h1, z2 = conv1_f(a, z1, deg, degc, b1.reshape(1, f), W2)
    h2, xn = conv2_f(a, z2, deg, degc, b2.reshape(1, f))
    v = lx_f(a, h2, deg)
    leader = nmax_f(a, v.reshape(1, n), v)

    lead_idx, pos, cnt = _sc_compact(leader.reshape(n), n)
    xl = _sc_gather(lead_idx, xn)

    cntf = cnt[:1].astype(jnp.float32).reshape(1, 1)
    s_mat = cos_f(xn, xl, leader, pos.reshape(n, 1), cntf)
    t_mat = tpass_f(a, s_mat)

    out = poolfinal_f(t_mat, s_mat, h2, cntf,
                      W3, b3.reshape(1, f), W4, b4.reshape(1, f),
                      Wr1, br1.reshape(1, f), Wr2, br2.reshape(1, f),
                      Wo, bo.reshape(1, 1))
    return out


# speedup vs baseline: 9.4555x; 1.0243x over previous
import functools

import jax
import jax.numpy as jnp
from jax import lax
from jax.experimental import pallas as pl
from jax.experimental.pallas import tpu as pltpu
from jax.experimental.pallas import tpu_sc as plsc

BM = 256
LMAX = 256


def _degz_body(a_ref, x_ref, w_ref, deg_ref, z_ref):
    deg_ref[...] = jnp.sum(a_ref[...], axis=1, keepdims=True)
    z_ref[...] = jnp.dot(x_ref[...], w_ref[...],
                         preferred_element_type=jnp.float32)


def _eye_full(n):
    rowg = lax.broadcasted_iota(jnp.int32, (n, n), 0)
    colg = lax.broadcasted_iota(jnp.int32, (n, n), 1)
    return rowg == colg


def _conv1_body(a_ref, zf_ref, disr_ref, disc_ref, b_ref, w2_ref,
                h_ref, z2_ref):
    a = a_ref[...]
    n = a.shape[0]
    eye = _eye_full(n).astype(jnp.float32)
    lap = ((a + eye) * disr_ref[...]) * disc_ref[...]
    acc = jnp.dot(lap, zf_ref[...], preferred_element_type=jnp.float32)
    h = jnp.maximum(acc + b_ref[...], 0.0)
    h_ref[...] = h
    z2_ref[...] = jnp.dot(h, w2_ref[...], preferred_element_type=jnp.float32)


def _conv2_body(a_ref, zf_ref, disr_ref, disc_ref, b_ref, h_ref, xn_ref):
    a = a_ref[...]
    n = a.shape[0]
    eye = _eye_full(n).astype(jnp.float32)
    lap = ((a + eye) * disr_ref[...]) * disc_ref[...]
    acc = jnp.dot(lap, zf_ref[...], preferred_element_type=jnp.float32)
    h = jnp.maximum(acc + b_ref[...], 0.0)
    h_ref[...] = h
    xn_ref[...] = h / jnp.sqrt(jnp.sum(h * h, axis=1, keepdims=True) + 1e-12)


def _lx_body(a_ref, hf_ref, degr_ref, v_ref):
    a = a_ref[...]
    n = a.shape[0]
    eyem = _eye_full(n)
    lapc = jnp.where(eyem, degr_ref[...], 0.0) - a
    w = jnp.dot(lapc, hf_ref[...], preferred_element_type=jnp.float32)
    v_ref[...] = jnp.sqrt(jnp.sum(w * w, axis=1, keepdims=True) + 1e-12)


def _nmax_body(a_ref, vc_ref, vb_ref, lead_ref):
    nm = jnp.max(a_ref[...] * vc_ref[...], axis=1, keepdims=True)
    lead_ref[...] = (vb_ref[...] >= nm).astype(jnp.int32)


def _cos_body(xn_ref, xl_ref, lead_ref, pos_ref, cnt_ref, s_ref):
    z = lax.dot_general(xn_ref[...], xl_ref[...],
                        (((1,), (1,)), ((), ())),
                        preferred_element_type=jnp.float32)
    m, n = z.shape
    coli = lax.broadcasted_iota(jnp.int32, (m, n), 1)
    z = jnp.where(coli.astype(jnp.float32) < cnt_ref[...], z, -1e9)
    zmax = jnp.max(z, axis=1, keepdims=True)
    lo = zmax - 1.0
    hi = zmax

    def bis(_, lh):
        lo, hi = lh
        mid = 0.5 * (lo + hi)
        fs = jnp.sum(jnp.maximum(z - mid, 0.0), axis=1, keepdims=True)
        ok = fs >= 1.0
        return jnp.where(ok, mid, lo), jnp.where(ok, hi, mid)

    lo, hi = lax.fori_loop(0, 14, bis, (lo, hi))
    tau = lo
    for _ in range(3):
        sup = z > tau
        kk = jnp.sum(sup.astype(jnp.float32), axis=1, keepdims=True)
        ss = jnp.sum(jnp.where(sup, z, 0.0), axis=1, keepdims=True)
        tau = (ss - 1.0) / jnp.maximum(kk, 1.0)
    s_mat = jnp.maximum(z - tau, 0.0)
    onehot = (coli == pos_ref[...]).astype(jnp.float32)
    s_ref[...] = jnp.where(lead_ref[...] > 0, onehot, s_mat)


def _t_body(s_ref, a_ref, t_ref):
    t_ref[...] = lax.dot_general(s_ref[...], a_ref[...],
                                 (((0,), (0,)), ((), ())),
                                 preferred_element_type=jnp.float32)


def _poolfinal_body(t_ref, s_ref, h_ref, cnt_ref,
                    w3_ref, b3_ref, w4_ref, b4_ref,
                    wr1_ref, br1_ref, wr2_ref, br2_ref, wo_ref, bo_ref,
                    out_ref):
    ap = jnp.dot(t_ref[...], s_ref[...], preferred_element_type=jnp.float32)
    xp = lax.dot_general(s_ref[...], h_ref[...], (((0,), (0,)), ((), ())),
                         preferred_element_type=jnp.float32)
    l = ap.shape[0]
    rid = lax.broadcasted_iota(jnp.int32, (l, l), 0)
    cid = lax.broadcasted_iota(jnp.int32, (l, l), 1)
    ah = ap + (rid == cid).astype(jnp.float32)
    deg_r = jnp.sum(ah, axis=1, keepdims=True)
    deg_c = jnp.sum(ah, axis=0, keepdims=True)
    lap = ((ah * (1.0 / jnp.sqrt(deg_r))) * (1.0 / jnp.sqrt(deg_c)))

    def conv(xv, w, b):
        z = jnp.dot(xv, w, preferred_element_type=jnp.float32)
        t = jnp.dot(lap, z, preferred_element_type=jnp.float32)
        return jnp.maximum(t + b, 0.0)

    x3 = conv(xp, w3_ref[...], b3_ref[...])
    x4 = conv(x3, w4_ref[...], b4_ref[...])
    ridf = lax.broadcasted_iota(jnp.int32, (l, 1), 0).astype(jnp.float32)
    msk = ridf < cnt_ref[...]
    pooled = jnp.sum(jnp.where(msk, x4, 0.0), axis=0, keepdims=True)
    pooled = pooled / (cnt_ref[...] + 1e-12)
    h = jnp.dot(pooled, wr1_ref[...],
                preferred_element_type=jnp.float32) + br1_ref[...]
    h = jnp.dot(h, wr2_ref[...],
                preferred_element_type=jnp.float32) + br2_ref[...]
    out_ref[...] = jnp.dot(h, wo_ref[...],
                           preferred_element_type=jnp.float32) + bo_ref[...]


def _blk(bm, bn, imap):
    return pl.BlockSpec((bm, bn), imap)


def _row(i):
    return (i, 0)


def _fix(*_):
    return (0, 0)


def _run_tc(n, f):
    g = n // BM
    f32 = jnp.float32

    def degz(a, x, w1):
        return pl.pallas_call(
            _degz_body, grid=(g,),
            in_specs=[_blk(BM, n, _row), _blk(BM, f, _row), _blk(f, f, _fix)],
            out_specs=[_blk(BM, 1, _row), _blk(BM, f, _row)],
            out_shape=[jax.ShapeDtypeStruct((n, 1), f32),
                       jax.ShapeDtypeStruct((n, f), f32)],
        )(a, x, w1)

    big = pltpu.CompilerParams(vmem_limit_bytes=100 * 1024 * 1024)

    def conv1(a, z, disr, disc, b, w2):
        return pl.pallas_call(
            _conv1_body,
            in_specs=[_blk(n, n, _fix), _blk(n, f, _fix), _blk(n, 1, _fix),
                      _blk(1, n, _fix), _blk(1, f, _fix), _blk(f, f, _fix)],
            out_specs=[_blk(n, f, _fix), _blk(n, f, _fix)],
            out_shape=[jax.ShapeDtypeStruct((n, f), f32),
                       jax.ShapeDtypeStruct((n, f), f32)],
            compiler_params=big,
        )(a, z, disr, disc, b, w2)

    def conv2(a, z, disr, disc, b):
        return pl.pallas_call(
            _conv2_body,
            in_specs=[_blk(n, n, _fix), _blk(n, f, _fix), _blk(n, 1, _fix),
                      _blk(1, n, _fix), _blk(1, f, _fix)],
            out_specs=[_blk(n, f, _fix), _blk(n, f, _fix)],
            out_shape=[jax.ShapeDtypeStruct((n, f), f32),
                       jax.ShapeDtypeStruct((n, f), f32)],
            compiler_params=big,
        )(a, z, disr, disc, b)

    def lx(a, h, degr):
        return pl.pallas_call(
            _lx_body,
            in_specs=[_blk(n, n, _fix), _blk(n, f, _fix), _blk(n, 1, _fix)],
            out_specs=_blk(n, 1, _fix),
            out_shape=jax.ShapeDtypeStruct((n, 1), f32),
            compiler_params=big,
        )(a, h, degr)

    def nmax(a, vcol, vblk):
        return pl.pallas_call(
            _nmax_body, grid=(g,),
            in_specs=[_blk(BM, n, _row), _blk(1, n, _fix), _blk(BM, 1, _row)],
            out_specs=_blk(BM, 1, _row),
            out_shape=jax.ShapeDtypeStruct((n, 1), jnp.int32),
        )(a, vcol, vblk)

    def cos(xn, xl, leadc, posc, cntf):
        return pl.pallas_call(
            _cos_body, grid=(g,),
            in_specs=[_blk(BM, f, _row), _blk(LMAX, f, _fix),
                      _blk(BM, 1, _row), _blk(BM, 1, _row), _blk(1, 1, _fix)],
            out_specs=_blk(BM, LMAX, _row),
            out_shape=jax.ShapeDtypeStruct((n, LMAX), f32),
        )(xn, xl, leadc, posc, cntf)

    def tpass(a, s):
        return pl.pallas_call(
            _t_body, grid=(g,),
            in_specs=[_blk(n, LMAX, _fix),
                      pl.BlockSpec((n, BM), lambda i: (0, i))],
            out_specs=pl.BlockSpec((LMAX, BM), lambda i: (0, i)),
            out_shape=jax.ShapeDtypeStruct((LMAX, n), f32),
        )(s, a)

    def poolfinal(t, s, h, cntf, w3, b3, w4, b4, wr1, br1, wr2, br2, wo, bo):
        return pl.pallas_call(
            _poolfinal_body,
            in_specs=[_blk(LMAX, n, _fix), _blk(n, LMAX, _fix),
                      _blk(n, f, _fix), _blk(1, 1, _fix),
                      _blk(f, f, _fix), _blk(1, f, _fix),
                      _blk(f, f, _fix), _blk(1, f, _fix), _blk(f, f, _fix),
                      _blk(1, f, _fix), _blk(f, f, _fix), _blk(1, f, _fix),
                      _blk(f, 1, _fix), _blk(1, 1, _fix)],
            out_specs=_blk(1, 1, _fix),
            out_shape=jax.ShapeDtypeStruct((1, 1), f32),
        )(t, s, h, cntf, w3, b3, w4, b4, wr1, br1, wr2, br2, wo, bo)

    return degz, conv1, conv2, lx, nmax, cos, tpass, poolfinal


def _sc_compact(leader, n):
    mesh = plsc.VectorSubcoreMesh(core_axis_name="c", subcore_axis_name="s")

    @functools.partial(
        pl.kernel, mesh=mesh,
        out_type=[jax.ShapeDtypeStruct((LMAX,), jnp.int32),
                  jax.ShapeDtypeStruct((n,), jnp.int32),
                  jax.ShapeDtypeStruct((16,), jnp.int32)],
        scratch_types=[pltpu.VMEM((n,), jnp.int32),
                       pltpu.VMEM((n,), jnp.int32),
                       pltpu.VMEM((LMAX,), jnp.int32),
                       pltpu.VMEM((16,), jnp.int32)],
        compiler_params=pltpu.CompilerParams(needs_layout_passes=False),
    )
    def k(lead_hbm, idx_hbm, pos_hbm, cnt_hbm, flags_v, pos_v, idx_v, cnt_v):
        wid = lax.axis_index("s") * 2 + lax.axis_index("c")

        @pl.when(wid == 0)
        def _():
            pltpu.sync_copy(lead_hbm, flags_v)

            def zstep(t, carry):
                idx_v[pl.ds(t * 16, 16)] = jnp.zeros((16,), jnp.int32)
                return carry

            lax.fori_loop(0, LMAX // 16, zstep, 0)

            def step(t, carry):
                fl = flags_v[pl.ds(t * 16, 16)]
                cs = plsc.cumsum(fl)
                pos = carry + cs - 1
                pos_v[pl.ds(t * 16, 16)] = pos
                posc = jnp.minimum(jnp.maximum(pos, 0), LMAX - 1)
                vals = lax.iota(jnp.int32, 16) + t * 16
                plsc.store_scatter(idx_v, [posc], vals, mask=fl > 0)
                return carry + jnp.sum(fl)

            total = lax.fori_loop(0, n // 16, step, 0)
            cnt_v[...] = jnp.broadcast_to(total, (16,))
            pltpu.sync_copy(idx_v, idx_hbm)
            pltpu.sync_copy(pos_v, pos_hbm)
            pltpu.sync_copy(cnt_v, cnt_hbm)

    return k(leader)


def _sc_gather(lead_idx, xn):
    f = xn.shape[1]
    bpw = LMAX // 32
    mesh = plsc.VectorSubcoreMesh(core_axis_name="c", subcore_axis_name="s")

    @functools.partial(
        pl.kernel, mesh=mesh,
        out_type=jax.ShapeDtypeStruct((LMAX, f), jnp.float32),
        scratch_types=[pltpu.VMEM((bpw,), jnp.int32),
                       pltpu.VMEM((bpw, f), jnp.float32),
                       pltpu.SemaphoreType.DMA],
    )
    def k(idx_hbm, xn_hbm, out_hbm, idx_v, rows_v, sem):
        wid = lax.axis_index("s") * 2 + lax.axis_index("c")
        base = wid * bpw
        pltpu.sync_copy(idx_hbm.at[pl.ds(base, bpw)], idx_v)
        pltpu.async_copy(xn_hbm.at[idx_v], rows_v, sem).wait()
        pltpu.sync_copy(rows_v, out_hbm.at[pl.ds(base, bpw)])

    return k(lead_idx, xn)


def kernel(x, a, i, W1, b1, W2, b2, W3, b3, W4, b4,
           Wr1, br1, Wr2, br2, Wo, bo):
    n, f = x.shape
    degz_f, conv1_f, conv2_f, lx_f, nmax_f, cos_f, tpass_f, poolfinal_f = \
        _run_tc(n, f)

    deg, z1 = degz_f(a, x, W1)
    degh = deg + 1.0
    dis = jnp.where(degh > 0, 1.0 / jnp.sqrt(degh), 0.0)
    disc = dis.reshape(1, n)
    h1, z2 = conv1_f(a, z1, dis, disc, b1.reshape(1, f), W2)
    h2, xn = conv2_f(a, z2, dis, disc, b2.reshape(1, f))
    v = lx_f(a, h2, deg)
    leader = nmax_f(a, v.reshape(1, n), v)

    lead_idx, pos, cnt = _sc_compact(leader.reshape(n), n)
    xl = _sc_gather(lead_idx, xn)

    cntf = cnt[:1].astype(jnp.float32).reshape(1, 1)
    s_mat = cos_f(xn, xl, leader, pos.reshape(n, 1), cntf)
    t_mat = tpass_f(a, s_mat)

    out = poolfinal_f(t_mat, s_mat, h2, cntf,
                      W3, b3.reshape(1, f), W4, b4.reshape(1, f),
                      Wr1, br1.reshape(1, f), Wr2, br2.reshape(1, f),
                      Wo, bo.reshape(1, 1))
    return out
